# baseline (speedup 1.0000x reference)
import functools

import jax
import jax.numpy as jnp
from jax import lax
from jax.experimental import pallas as pl
from jax.experimental.pallas import tpu as pltpu
from jax.experimental.pallas import tpu_sc as plsc

_NUM_REGIONS = 116
_RPAD = 128
_LANES = 16
_NC, _NS = 2, 16
_NW = _NC * _NS
_EPS = 1e-6
_GAMMA = 0.001
_COLS = 512


@functools.partial(jax.jit, static_argnums=(3,))
def _sc_hist(real_2d, fake_2d, ids_2d, chunk_rows):
    rows = real_2d.shape[0]
    rows_per_w = rows // _NW
    n_chunks = rows_per_w // chunk_rows
    groups = chunk_rows * _COLS // _LANES
    mesh = plsc.VectorSubcoreMesh(core_axis_name="c", subcore_axis_name="s")

    @functools.partial(
        pl.kernel,
        mesh=mesh,
        out_type=[
            jax.ShapeDtypeStruct((_NW * _LANES, _RPAD), jnp.float32),
            jax.ShapeDtypeStruct((_NW * _LANES, _RPAD), jnp.float32),
        ],
        scratch_types=[
            pltpu.VMEM((2 * chunk_rows, _COLS), jnp.float32),
            pltpu.VMEM((2 * chunk_rows, _COLS), jnp.float32),
            pltpu.VMEM((2 * chunk_rows, _COLS), jnp.int32),
            pltpu.VMEM((_LANES, _RPAD), jnp.float32),
            pltpu.VMEM((_LANES, _RPAD), jnp.float32),
            pltpu.SemaphoreType.DMA,
            pltpu.SemaphoreType.DMA,
        ],
        compiler_params=pltpu.CompilerParams(needs_layout_passes=False),
    )
    def hist(real_hbm, fake_hbm, ids_hbm, out_s, out_c,
             rbuf, fbuf, ibuf, accs, accc, sem0, sem1):
        cid = lax.axis_index("c")
        sid = lax.axis_index("s")
        wid = sid * _NC + cid
        base = wid * rows_per_w
        sems = (sem0, sem1)

        zeros16 = jnp.zeros((_LANES,), jnp.float32)

        def zero_body(i, _):
            def zrow(j, _):
                accs[i, pl.ds(j * _LANES, _LANES)] = zeros16
                accc[i, pl.ds(j * _LANES, _LANES)] = zeros16
                return 0
            lax.fori_loop(0, _RPAD // _LANES, zrow, 0)
            return 0

        lax.fori_loop(0, _LANES, zero_body, 0)

        lane_iota = lax.iota(jnp.int32, _LANES)
        ones16 = jnp.ones((_LANES,), jnp.float32)

        def start_fetch(ci, slot):
            r0 = base + ci * chunk_rows
            b0 = slot * chunk_rows
            pltpu.async_copy(real_hbm.at[pl.ds(r0, chunk_rows), :],
                             rbuf.at[pl.ds(b0, chunk_rows), :], sems[slot])
            pltpu.async_copy(fake_hbm.at[pl.ds(r0, chunk_rows), :],
                             fbuf.at[pl.ds(b0, chunk_rows), :], sems[slot])
            pltpu.async_copy(ids_hbm.at[pl.ds(r0, chunk_rows), :],
                             ibuf.at[pl.ds(b0, chunk_rows), :], sems[slot])

        def wait_fetch(ci, slot):
            r0 = base + ci * chunk_rows
            b0 = slot * chunk_rows
            pltpu.make_async_copy(real_hbm.at[pl.ds(r0, chunk_rows), :],
                                  rbuf.at[pl.ds(b0, chunk_rows), :],
                                  sems[slot]).wait()
            pltpu.make_async_copy(fake_hbm.at[pl.ds(r0, chunk_rows), :],
                                  fbuf.at[pl.ds(b0, chunk_rows), :],
                                  sems[slot]).wait()
            pltpu.make_async_copy(ids_hbm.at[pl.ds(r0, chunk_rows), :],
                                  ibuf.at[pl.ds(b0, chunk_rows), :],
                                  sems[slot]).wait()

        start_fetch(0, 0)

        gpr = _COLS // _LANES

        def chunk_body(ci, _):
            parity = lax.rem(ci, 2)

            @pl.when(jnp.logical_and(ci + 1 < n_chunks, parity == 0))
            def _():
                start_fetch(ci + 1, 1)

            @pl.when(jnp.logical_and(ci + 1 < n_chunks, parity == 1))
            def _():
                start_fetch(ci + 1, 0)

            @pl.when(parity == 0)
            def _():
                wait_fetch(ci, 0)

            @pl.when(parity == 1)
            def _():
                wait_fetch(ci, 1)

            base_row = parity * chunk_rows

            @plsc.parallel_loop(0, groups, unroll=8)
            def _(i):
                row = base_row + i // gpr
                col = (i % gpr) * _LANES
                r = rbuf[row, pl.ds(col, _LANES)]
                f = fbuf[row, pl.ds(col, _LANES)]
                ids = ibuf[row, pl.ds(col, _LANES)]
                d = jnp.abs(r - f)
                plsc.addupdate_scatter(accs, [lane_iota, ids], d)
                plsc.addupdate_scatter(accc, [lane_iota, ids], ones16)
            return 0

        lax.fori_loop(0, n_chunks, chunk_body, 0)

        pltpu.sync_copy(accs, out_s.at[pl.ds(wid * _LANES, _LANES), :])
        pltpu.sync_copy(accc, out_c.at[pl.ds(wid * _LANES, _LANES), :])

    return hist(real_2d, fake_2d, ids_2d)


def _epilogue_kernel(sp_ref, cp_ref, ml_ref, inv_n_ref, out_ref):
    sums = jnp.sum(sp_ref[...], axis=0)
    cnts = jnp.sum(cp_ref[...], axis=0)
    means = sums / (cnts + _EPS)
    mx = jnp.maximum(jnp.max(means), ml_ref[0])
    total = jnp.sum(sums)
    weighted = jnp.sum(means * sums)
    out_ref[0, 0] = (total + _GAMMA * weighted / (mx + _EPS)) * inv_n_ref[0]


def _epilogue(sp, cp, max_l1_loss, inv_n):
    return pl.pallas_call(
        _epilogue_kernel,
        out_shape=jax.ShapeDtypeStruct((1, 1), jnp.float32),
        in_specs=[
            pl.BlockSpec(memory_space=pltpu.VMEM),
            pl.BlockSpec(memory_space=pltpu.VMEM),
            pl.BlockSpec(memory_space=pltpu.SMEM),
            pl.BlockSpec(memory_space=pltpu.SMEM),
        ],
        out_specs=pl.BlockSpec(memory_space=pltpu.SMEM),
    )(sp, cp, max_l1_loss, inv_n)


def kernel(real, fake, region_map, regions, max_l1_loss):
    n = real.size
    rows = n // _COLS
    real_2d = real.reshape(rows, _COLS)
    fake_2d = fake.reshape(rows, _COLS)
    ids_2d = region_map.reshape(rows, _COLS)
    sp, cp = _sc_hist(real_2d, fake_2d, ids_2d, 16)
    inv_n = jnp.full((1,), 1.0 / n, dtype=jnp.float32)
    out = _epilogue(sp, cp, max_l1_loss, inv_n)
    return out[0, 0]


# speedup vs baseline: 1.0045x; 1.0045x over previous
import functools

import jax
import jax.numpy as jnp
from jax import lax
from jax.experimental import pallas as pl
from jax.experimental.pallas import tpu as pltpu
from jax.experimental.pallas import tpu_sc as plsc

_NUM_REGIONS = 116
_RPAD = 128
_LANES = 16
_NC, _NS = 2, 16
_NW = _NC * _NS
_EPS = 1e-6
_GAMMA = 0.001
_COLS = 512


@functools.partial(jax.jit, static_argnums=(3,))
def _sc_hist(real_2d, fake_2d, ids_2d, chunk_rows):
    rows = real_2d.shape[0]
    rows_per_w = rows // _NW
    n_chunks = rows_per_w // chunk_rows
    groups = chunk_rows * _COLS // _LANES
    mesh = plsc.VectorSubcoreMesh(core_axis_name="c", subcore_axis_name="s")

    @functools.partial(
        pl.kernel,
        mesh=mesh,
        out_type=[
            jax.ShapeDtypeStruct((_NW * _LANES, _RPAD), jnp.float32),
            jax.ShapeDtypeStruct((_NW * _LANES, _RPAD), jnp.float32),
        ],
        scratch_types=[
            pltpu.VMEM((2 * chunk_rows, _COLS), jnp.float32),
            pltpu.VMEM((2 * chunk_rows, _COLS), jnp.float32),
            pltpu.VMEM((2 * chunk_rows, _COLS), jnp.int32),
            pltpu.VMEM((_LANES, _RPAD), jnp.float32),
            pltpu.VMEM((_LANES, _RPAD), jnp.float32),
            pltpu.SemaphoreType.DMA,
            pltpu.SemaphoreType.DMA,
        ],
        compiler_params=pltpu.CompilerParams(needs_layout_passes=False),
    )
    def hist(real_hbm, fake_hbm, ids_hbm, out_s, out_c,
             rbuf, fbuf, ibuf, accs, accc, sem0, sem1):
        cid = lax.axis_index("c")
        sid = lax.axis_index("s")
        wid = sid * _NC + cid
        base = wid * rows_per_w
        sems = (sem0, sem1)

        def start_fetch(ci, slot):
            r0 = base + ci * chunk_rows
            b0 = slot * chunk_rows
            pltpu.async_copy(real_hbm.at[pl.ds(r0, chunk_rows), :],
                             rbuf.at[pl.ds(b0, chunk_rows), :], sems[slot])
            pltpu.async_copy(fake_hbm.at[pl.ds(r0, chunk_rows), :],
                             fbuf.at[pl.ds(b0, chunk_rows), :], sems[slot])
            pltpu.async_copy(ids_hbm.at[pl.ds(r0, chunk_rows), :],
                             ibuf.at[pl.ds(b0, chunk_rows), :], sems[slot])

        start_fetch(0, 0)

        zeros16 = jnp.zeros((_LANES,), jnp.float32)

        def zero_body(i, _):
            def zrow(j, _):
                accs[i, pl.ds(j * _LANES, _LANES)] = zeros16
                accc[i, pl.ds(j * _LANES, _LANES)] = zeros16
                return 0
            lax.fori_loop(0, _RPAD // _LANES, zrow, 0)
            return 0

        lax.fori_loop(0, _LANES, zero_body, 0)

        lane_iota = lax.iota(jnp.int32, _LANES)
        ones16 = jnp.ones((_LANES,), jnp.float32)

        def wait_fetch(ci, slot):
            r0 = base + ci * chunk_rows
            b0 = slot * chunk_rows
            pltpu.make_async_copy(real_hbm.at[pl.ds(r0, chunk_rows), :],
                                  rbuf.at[pl.ds(b0, chunk_rows), :],
                                  sems[slot]).wait()
            pltpu.make_async_copy(fake_hbm.at[pl.ds(r0, chunk_rows), :],
                                  fbuf.at[pl.ds(b0, chunk_rows), :],
                                  sems[slot]).wait()
            pltpu.make_async_copy(ids_hbm.at[pl.ds(r0, chunk_rows), :],
                                  ibuf.at[pl.ds(b0, chunk_rows), :],
                                  sems[slot]).wait()

        gpr = _COLS // _LANES

        def chunk_body(ci, _):
            parity = lax.rem(ci, 2)

            @pl.when(jnp.logical_and(ci + 1 < n_chunks, parity == 0))
            def _():
                start_fetch(ci + 1, 1)

            @pl.when(jnp.logical_and(ci + 1 < n_chunks, parity == 1))
            def _():
                start_fetch(ci + 1, 0)

            @pl.when(parity == 0)
            def _():
                wait_fetch(ci, 0)

            @pl.when(parity == 1)
            def _():
                wait_fetch(ci, 1)

            base_row = parity * chunk_rows

            @plsc.parallel_loop(0, groups, unroll=8)
            def _(i):
                row = base_row + i // gpr
                col = (i % gpr) * _LANES
                r = rbuf[row, pl.ds(col, _LANES)]
                f = fbuf[row, pl.ds(col, _LANES)]
                ids = ibuf[row, pl.ds(col, _LANES)]
                d = jnp.abs(r - f)
                plsc.addupdate_scatter(accs, [lane_iota, ids], d)
                plsc.addupdate_scatter(accc, [lane_iota, ids], ones16)
            return 0

        lax.fori_loop(0, n_chunks, chunk_body, 0)

        pltpu.sync_copy(accs, out_s.at[pl.ds(wid * _LANES, _LANES), :])
        pltpu.sync_copy(accc, out_c.at[pl.ds(wid * _LANES, _LANES), :])

    return hist(real_2d, fake_2d, ids_2d)


def _epilogue_kernel(sp_ref, cp_ref, ml_ref, inv_n_ref, out_ref):
    sums = jnp.sum(sp_ref[...], axis=0)
    cnts = jnp.sum(cp_ref[...], axis=0)
    means = sums / (cnts + _EPS)
    mx = jnp.maximum(jnp.max(means), ml_ref[0])
    total = jnp.sum(sums)
    weighted = jnp.sum(means * sums)
    out_ref[0, 0] = (total + _GAMMA * weighted / (mx + _EPS)) * inv_n_ref[0]


def _epilogue(sp, cp, max_l1_loss, inv_n):
    return pl.pallas_call(
        _epilogue_kernel,
        out_shape=jax.ShapeDtypeStruct((1, 1), jnp.float32),
        in_specs=[
            pl.BlockSpec(memory_space=pltpu.VMEM),
            pl.BlockSpec(memory_space=pltpu.VMEM),
            pl.BlockSpec(memory_space=pltpu.SMEM),
            pl.BlockSpec(memory_space=pltpu.SMEM),
        ],
        out_specs=pl.BlockSpec(memory_space=pltpu.SMEM),
    )(sp, cp, max_l1_loss, inv_n)


def kernel(real, fake, region_map, regions, max_l1_loss):
    n = real.size
    rows = n // _COLS
    real_2d = real.reshape(rows, _COLS)
    fake_2d = fake.reshape(rows, _COLS)
    ids_2d = region_map.reshape(rows, _COLS)
    sp, cp = _sc_hist(real_2d, fake_2d, ids_2d, 16)
    inv_n = jnp.full((1,), 1.0 / n, dtype=jnp.float32)
    out = _epilogue(sp, cp, max_l1_loss, inv_n)
    return out[0, 0]
